# baseline (speedup 1.0000x reference)
import jax
import jax.numpy as jnp
from jax import lax
from jax.experimental import pallas as pl
from jax.experimental.pallas import tpu as pltpu
from jax.experimental.pallas import tpu_sc as plsc

WORD = 1000000
D = 64
B = 16384
NNEG = 5

NC = 2
NS = 16
NW = NC * NS
BPW = B // NW
CHUNK = 128
NCHUNK = BPW // CHUNK
DV = D // 16


def _sc_body(u_hbm, v_hbm, posu_hbm, posv_hbm, negt_hbm, out1_hbm, out2_hbm,
             uidx, vidx, nidx, ubuf, vbuf, nbuf, accbuf, sem):
    wid = lax.axis_index("s") * NC + lax.axis_index("c")
    base = wid * BPW

    def dot_loop(abuf, bbuf, acc):
        def body(r, a):
            for q in range(DV):
                a = a + abuf[r, pl.ds(16 * q, 16)] * bbuf[r, pl.ds(16 * q, 16)]
            return a
        return lax.fori_loop(0, CHUNK, body, acc)

    acc1 = jnp.zeros((16,), jnp.float32)
    acc2 = jnp.zeros((16,), jnp.float32)
    for c in range(NCHUNK):
        off = base + c * CHUNK
        pltpu.sync_copy(posu_hbm.at[pl.ds(off, CHUNK)], uidx)
        pltpu.sync_copy(posv_hbm.at[pl.ds(off, CHUNK)], vidx)
        pltpu.async_copy(u_hbm.at[uidx], ubuf, sem).wait()
        pltpu.async_copy(v_hbm.at[vidx], vbuf, sem).wait()
        acc1 = dot_loop(ubuf, vbuf, acc1)
        for n in range(NNEG):
            pltpu.sync_copy(negt_hbm.at[pl.ds(n * B + off, CHUNK)], nidx)
            pltpu.async_copy(v_hbm.at[nidx], nbuf, sem).wait()
            acc2 = dot_loop(ubuf, nbuf, acc2)

    accbuf[...] = acc1
    pltpu.sync_copy(accbuf, out1_hbm.at[wid])
    accbuf[...] = acc2
    pltpu.sync_copy(accbuf, out2_hbm.at[wid])


@jax.jit
def _skipgram(u_table, v_table, pos_u, pos_v, neg_t):
    mesh = plsc.VectorSubcoreMesh(core_axis_name="c", subcore_axis_name="s")
    f = pl.kernel(
        _sc_body,
        out_type=(
            jax.ShapeDtypeStruct((NW, 16), jnp.float32),
            jax.ShapeDtypeStruct((NW, 16), jnp.float32),
        ),
        mesh=mesh,
        compiler_params=pltpu.CompilerParams(use_tc_tiling_on_sc=False),
        scratch_types=[
            pltpu.VMEM((CHUNK,), jnp.int32),
            pltpu.VMEM((CHUNK,), jnp.int32),
            pltpu.VMEM((CHUNK,), jnp.int32),
            pltpu.VMEM((CHUNK, D), jnp.float32),
            pltpu.VMEM((CHUNK, D), jnp.float32),
            pltpu.VMEM((CHUNK, D), jnp.float32),
            pltpu.VMEM((16,), jnp.float32),
            pltpu.SemaphoreType.DMA,
        ],
    )
    out1, out2 = f(u_table, v_table, pos_u, pos_v, neg_t)
    s1 = jnp.sum(out1)
    s2 = jnp.sum(out2)
    return -(jax.nn.log_sigmoid(s1) + jax.nn.log_sigmoid(-s2))


def kernel(u_table, v_table, pos_u, pos_v, neg_v):
    neg_t = neg_v.T.reshape(-1)
    return _skipgram(u_table, v_table, pos_u, pos_v, neg_t)


# speedup vs baseline: 1.0325x; 1.0325x over previous
import jax
import jax.numpy as jnp
from jax import lax
from jax.experimental import pallas as pl
from jax.experimental.pallas import tpu as pltpu
from jax.experimental.pallas import tpu_sc as plsc

WORD = 1000000
D = 64
B = 16384
NNEG = 5

NC = 2
NS = 16
NW = NC * NS
BPW = B // NW
CHUNK = 128
NCHUNK = BPW // CHUNK
DV = D // 16


def _sc_body(u_hbm, v_hbm, posu_hbm, posv_hbm, negw_hbm, out1_hbm, out2_hbm,
             uidx, vidx, nidx,
             ubuf0, vbuf0, nbuf00, nbuf01, nbuf02, nbuf03, nbuf04,
             ubuf1, vbuf1, nbuf10, nbuf11, nbuf12, nbuf13, nbuf14,
             accbuf, sem0, sem1):
    wid = lax.axis_index("s") * NC + lax.axis_index("c")
    bufs = [
        (ubuf0, vbuf0, [nbuf00, nbuf01, nbuf02, nbuf03, nbuf04], sem0),
        (ubuf1, vbuf1, [nbuf10, nbuf11, nbuf12, nbuf13, nbuf14], sem1),
    ]

    pltpu.sync_copy(posu_hbm.at[wid], uidx)
    pltpu.sync_copy(posv_hbm.at[wid], vidx)
    pltpu.sync_copy(negw_hbm.at[wid], nidx)

    def fire(c, p):
        ub, vb, nb, sem = bufs[p]
        ds = [pltpu.async_copy(u_hbm.at[uidx.at[c]], ub, sem),
              pltpu.async_copy(v_hbm.at[vidx.at[c]], vb, sem)]
        for n in range(NNEG):
            ds.append(pltpu.async_copy(v_hbm.at[nidx.at[n, c]], nb[n], sem))
        return ds

    def compute(p, accs):
        ub, vb, nb, _ = bufs[p]

        def body(r, a):
            a1, a20, a21, a22, a23, a24 = a
            for q in range(DV):
                s = pl.ds(16 * q, 16)
                u = ub[r, s]
                a1 = a1 + u * vb[r, s]
                a20 = a20 + u * nb[0][r, s]
                a21 = a21 + u * nb[1][r, s]
                a22 = a22 + u * nb[2][r, s]
                a23 = a23 + u * nb[3][r, s]
                a24 = a24 + u * nb[4][r, s]
            return (a1, a20, a21, a22, a23, a24)

        return lax.fori_loop(0, CHUNK, body, accs)

    z = jnp.zeros((16,), jnp.float32)
    accs = (z, z, z, z, z, z)
    inflight = fire(0, 0)
    for c in range(NCHUNK):
        p = c % 2
        for d in inflight:
            d.wait()
        if c + 1 < NCHUNK:
            inflight = fire(c + 1, 1 - p)
        accs = compute(p, accs)

    accbuf[...] = accs[0]
    pltpu.sync_copy(accbuf, out1_hbm.at[wid])
    accbuf[...] = accs[1] + accs[2] + accs[3] + accs[4] + accs[5]
    pltpu.sync_copy(accbuf, out2_hbm.at[wid])


@jax.jit
def _skipgram(u_table, v_table, pos_u, pos_v, neg_w):
    mesh = plsc.VectorSubcoreMesh(core_axis_name="c", subcore_axis_name="s")
    row = pltpu.VMEM((CHUNK, D), jnp.float32)
    f = pl.kernel(
        _sc_body,
        out_type=(
            jax.ShapeDtypeStruct((NW, 16), jnp.float32),
            jax.ShapeDtypeStruct((NW, 16), jnp.float32),
        ),
        mesh=mesh,
        compiler_params=pltpu.CompilerParams(use_tc_tiling_on_sc=False),
        scratch_types=[
            pltpu.VMEM((NCHUNK, CHUNK), jnp.int32),
            pltpu.VMEM((NCHUNK, CHUNK), jnp.int32),
            pltpu.VMEM((NNEG, NCHUNK, CHUNK), jnp.int32),
            row, row, row, row, row, row, row,
            row, row, row, row, row, row, row,
            pltpu.VMEM((16,), jnp.float32),
            pltpu.SemaphoreType.DMA,
            pltpu.SemaphoreType.DMA,
        ],
    )
    out1, out2 = f(u_table, v_table, pos_u, pos_v, neg_w)
    s1 = jnp.sum(out1)
    s2 = jnp.sum(out2)
    return -(jax.nn.log_sigmoid(s1) + jax.nn.log_sigmoid(-s2))


def kernel(u_table, v_table, pos_u, pos_v, neg_v):
    pos_u_w = pos_u.reshape(NW, NCHUNK, CHUNK)
    pos_v_w = pos_v.reshape(NW, NCHUNK, CHUNK)
    neg_w = neg_v.reshape(NW, NCHUNK, CHUNK, NNEG).transpose(0, 3, 1, 2)
    return _skipgram(u_table, v_table, pos_u_w, pos_v_w, neg_w)
